# baseline (speedup 1.0000x reference)
import jax
import jax.numpy as jnp
from jax.experimental import pallas as pl

_B = 1024
_F = 100
_D = 16
_V = 100
_TI = 256
_TR = _TI * _D


def _fm_kernel(x_ref, xt_ref, wcol_ref, m1_ref, eg_ref, eg2_ref, out_ref):
    xb = x_ref[...]
    iota = jax.lax.broadcasted_iota(jnp.int32, (1, 1, _V), 2)
    eq = (xb[:, :, None] == iota).astype(jnp.int32)
    cmat = jnp.sum(eq, axis=1).astype(jnp.float32)
    cexp = jnp.dot(m1_ref[...], cmat,
                   preferred_element_type=jnp.float32)
    se = jnp.sum(cexp * eg_ref[...], axis=1, keepdims=True)
    se2 = jnp.sum(cexp * eg2_ref[...], axis=1, keepdims=True)
    half_cross = 0.25 * (se * se) - 0.25 * se2
    lin_row = jnp.sum(wcol_ref[...] * xt_ref[...], axis=0, keepdims=True)
    half_lin = 0.5 * lin_row
    t = half_cross + half_lin
    out_ref[...] = 0.5 * jnp.tanh(t) + 0.5


def kernel(x, emb_table, linear_weights):
    n_i = _B // _TI
    xt = x.astype(jnp.float32).T
    wcol = linear_weights.reshape(_F, 1)
    m1 = jnp.repeat(jnp.eye(_TI, dtype=jnp.float32), _D, axis=0)
    eg = jnp.tile(emb_table.T, (_TI, 1))
    eg2 = eg * eg

    out2 = pl.pallas_call(
        _fm_kernel,
        grid=(n_i,),
        in_specs=[
            pl.BlockSpec((_TI, _F), lambda i: (i, 0)),
            pl.BlockSpec((_F, _B), lambda i: (0, 0)),
            pl.BlockSpec((_F, 1), lambda i: (0, 0)),
            pl.BlockSpec((_TR, _TI), lambda i: (0, 0)),
            pl.BlockSpec((_TR, _V), lambda i: (0, 0)),
            pl.BlockSpec((_TR, _V), lambda i: (0, 0)),
        ],
        out_specs=pl.BlockSpec((_TR, _B), lambda i: (i, 0)),
        out_shape=jax.ShapeDtypeStruct((_B * _D, _B), jnp.float32),
    )(x, xt, wcol, m1, eg, eg2)

    return out2.reshape(_B, _D, _B).transpose(0, 2, 1)


# speedup vs baseline: 1.1217x; 1.1217x over previous
import jax
import jax.numpy as jnp
from jax.experimental import pallas as pl

_B = 1024
_F = 100
_D = 16
_V = 100
_TI = 128
_TR = _TI * _D


def _fm_kernel(x_ref, xt_ref, wcol_ref, m1_ref, eg_ref, eg2_ref, out_ref):
    xb = x_ref[...]
    iota = jax.lax.broadcasted_iota(jnp.int32, (1, 1, _V), 2)
    eq = (xb[:, :, None] == iota).astype(jnp.int32)
    cmat = jnp.sum(eq, axis=1).astype(jnp.float32)
    cexp = jnp.dot(m1_ref[...], cmat,
                   preferred_element_type=jnp.float32)
    se = jnp.sum(cexp * eg_ref[...], axis=1, keepdims=True)
    se2 = jnp.sum(cexp * eg2_ref[...], axis=1, keepdims=True)
    half_cross = 0.25 * (se * se) - 0.25 * se2
    lin_row = jnp.sum(wcol_ref[...] * xt_ref[...], axis=0, keepdims=True)
    half_lin = 0.5 * lin_row
    t = half_cross + half_lin
    out_ref[...] = 0.5 * jnp.tanh(t) + 0.5


def kernel(x, emb_table, linear_weights):
    n_i = _B // _TI
    xt = x.astype(jnp.float32).T
    wcol = linear_weights.reshape(_F, 1)
    m1 = jnp.repeat(jnp.eye(_TI, dtype=jnp.float32), _D, axis=0)
    eg = jnp.tile(emb_table.T, (_TI, 1))
    eg2 = eg * eg

    out2 = pl.pallas_call(
        _fm_kernel,
        grid=(n_i,),
        in_specs=[
            pl.BlockSpec((_TI, _F), lambda i: (i, 0)),
            pl.BlockSpec((_F, _B), lambda i: (0, 0)),
            pl.BlockSpec((_F, 1), lambda i: (0, 0)),
            pl.BlockSpec((_TR, _TI), lambda i: (0, 0)),
            pl.BlockSpec((_TR, _V), lambda i: (0, 0)),
            pl.BlockSpec((_TR, _V), lambda i: (0, 0)),
        ],
        out_specs=pl.BlockSpec((_TR, _B), lambda i: (i, 0)),
        out_shape=jax.ShapeDtypeStruct((_B * _D, _B), jnp.float32),
    )(x, xt, wcol, m1, eg, eg2)

    return out2.reshape(_B, _D, _B).transpose(0, 2, 1)


# speedup vs baseline: 1.2198x; 1.0874x over previous
import jax
import jax.numpy as jnp
from jax.experimental import pallas as pl

_B = 1024
_F = 100
_D = 16
_V = 100
_TI = 128
_TR = _TI * _D


def _fm_kernel(x_ref, xt_ref, wcol_ref, et_ref, et2_ref, out_ref):
    xb = x_ref[...]
    iota = jax.lax.broadcasted_iota(jnp.int32, (1, 1, _V), 2)
    eq = (xb[:, :, None] == iota).astype(jnp.int32)
    cmat = jnp.sum(eq, axis=1).astype(jnp.float32)
    cexp = jnp.broadcast_to(cmat[:, None, :], (_TI, _D, _V)).reshape(_TR, _V)
    eg = jnp.broadcast_to(et_ref[...][None, :, :], (_TI, _D, _V)).reshape(_TR, _V)
    eg2 = jnp.broadcast_to(et2_ref[...][None, :, :], (_TI, _D, _V)).reshape(_TR, _V)
    se = jnp.sum(cexp * eg, axis=1, keepdims=True)
    se2 = jnp.sum(cexp * eg2, axis=1, keepdims=True)
    half_cross = 0.25 * (se * se) - 0.25 * se2
    lin_row = jnp.sum(wcol_ref[...] * xt_ref[...], axis=0, keepdims=True)
    half_lin = 0.5 * lin_row
    t = half_cross + half_lin
    out_ref[...] = 0.5 * jnp.tanh(t) + 0.5


def kernel(x, emb_table, linear_weights):
    n_i = _B // _TI
    xt = x.astype(jnp.float32).T
    wcol = linear_weights.reshape(_F, 1)
    et = emb_table.T
    et2 = et * et

    out2 = pl.pallas_call(
        _fm_kernel,
        grid=(n_i,),
        in_specs=[
            pl.BlockSpec((_TI, _F), lambda i: (i, 0)),
            pl.BlockSpec((_F, _B), lambda i: (0, 0)),
            pl.BlockSpec((_F, 1), lambda i: (0, 0)),
            pl.BlockSpec((_D, _V), lambda i: (0, 0)),
            pl.BlockSpec((_D, _V), lambda i: (0, 0)),
        ],
        out_specs=pl.BlockSpec((_TR, _B), lambda i: (i, 0)),
        out_shape=jax.ShapeDtypeStruct((_B * _D, _B), jnp.float32),
    )(x, xt, wcol, et, et2)

    return out2.reshape(_B, _D, _B).transpose(0, 2, 1)


# speedup vs baseline: 1.2608x; 1.0336x over previous
import jax
import jax.numpy as jnp
from jax.experimental import pallas as pl

_B = 1024
_F = 100
_D = 16
_V = 100
_TI = 64
_TR = _TI * _D


def _fm_kernel(x_ref, xt_ref, wcol_ref, et_ref, et2_ref, out_ref):
    xb = x_ref[...]
    iota = jax.lax.broadcasted_iota(jnp.int32, (1, 1, _V), 2)
    eq = (xb[:, :, None] == iota).astype(jnp.int32)
    cmat = jnp.sum(eq, axis=1).astype(jnp.float32)
    cexp = jnp.broadcast_to(cmat[:, None, :], (_TI, _D, _V)).reshape(_TR, _V)
    eg = jnp.broadcast_to(et_ref[...][None, :, :], (_TI, _D, _V)).reshape(_TR, _V)
    eg2 = jnp.broadcast_to(et2_ref[...][None, :, :], (_TI, _D, _V)).reshape(_TR, _V)
    se = jnp.sum(cexp * eg, axis=1, keepdims=True)
    se2 = jnp.sum(cexp * eg2, axis=1, keepdims=True)
    half_cross = 0.25 * (se * se) - 0.25 * se2
    lin_row = jnp.sum(wcol_ref[...] * xt_ref[...], axis=0, keepdims=True)
    half_lin = 0.5 * lin_row
    t = half_cross + half_lin
    out_ref[...] = 0.5 * jnp.tanh(t) + 0.5


def kernel(x, emb_table, linear_weights):
    n_i = _B // _TI
    xt = x.astype(jnp.float32).T
    wcol = linear_weights.reshape(_F, 1)
    et = emb_table.T
    et2 = et * et

    out2 = pl.pallas_call(
        _fm_kernel,
        grid=(n_i,),
        in_specs=[
            pl.BlockSpec((_TI, _F), lambda i: (i, 0)),
            pl.BlockSpec((_F, _B), lambda i: (0, 0)),
            pl.BlockSpec((_F, 1), lambda i: (0, 0)),
            pl.BlockSpec((_D, _V), lambda i: (0, 0)),
            pl.BlockSpec((_D, _V), lambda i: (0, 0)),
        ],
        out_specs=pl.BlockSpec((_TR, _B), lambda i: (i, 0)),
        out_shape=jax.ShapeDtypeStruct((_B * _D, _B), jnp.float32),
    )(x, xt, wcol, et, et2)

    return out2.reshape(_B, _D, _B).transpose(0, 2, 1)
